# Initial kernel scaffold; baseline (speedup 1.0000x reference)
import jax
import jax.numpy as jnp
from jax.experimental import pallas as pl


def kernel(positions, pe):
    raise NotImplementedError("write your pallas kernel here")


# speedup vs baseline: 1.9801x; 1.9801x over previous
import functools

import jax
import jax.numpy as jnp
from jax import lax
from jax.experimental import pallas as pl
from jax.experimental.pallas import tpu as pltpu
from jax.experimental.pallas import tpu_sc as plsc

_LENGTH = 8192
_EMBED = 1024
_BATCH = 4
_SEQ = 8192
_NTOT = _BATCH * _SEQ

_NC = 2
_NS = 16
_NW = _NC * _NS
_B_PER_W = _NTOT // _NW
_CH = 32
_NCHUNK = _B_PER_W // _CH

_mesh = plsc.VectorSubcoreMesh(core_axis_name="c", subcore_axis_name="s")


@functools.partial(
    pl.kernel,
    mesh=_mesh,
    out_type=jax.ShapeDtypeStruct((_NTOT, _EMBED), jnp.float32),
    scratch_types=[
        pltpu.VMEM((_B_PER_W,), jnp.int32),
        pltpu.VMEM((2, _CH, _EMBED), jnp.float32),
        pltpu.SemaphoreType.DMA,
    ],
)
def _sc_gather(pe_hbm, idx_hbm, out_hbm, idx_v, rows_v, gsem):
    wid = lax.axis_index("s") * _NC + lax.axis_index("c")
    base = wid * _B_PER_W
    pltpu.sync_copy(idx_hbm.at[pl.ds(base, _B_PER_W)], idx_v)

    def body(j, _):
        pltpu.async_copy(
            pe_hbm.at[idx_v.at[pl.ds(j * _CH, _CH)]], rows_v.at[0], gsem
        ).wait()
        pltpu.sync_copy(rows_v.at[0], out_hbm.at[pl.ds(base + j * _CH, _CH)])
        return 0

    lax.fori_loop(0, _NCHUNK, body, 0)


def kernel(positions, pe):
    idx = positions.reshape(-1).astype(jnp.int32)
    out = _sc_gather(pe, idx)
    return out.reshape(_BATCH, _SEQ, _EMBED)


# speedup vs baseline: 2.3757x; 1.1998x over previous
import functools

import jax
import jax.numpy as jnp
from jax import lax
from jax.experimental import pallas as pl
from jax.experimental.pallas import tpu as pltpu
from jax.experimental.pallas import tpu_sc as plsc

_LENGTH = 8192
_EMBED = 1024
_BATCH = 4
_SEQ = 8192
_NTOT = _BATCH * _SEQ

_NC = 2
_NS = 16
_NW = _NC * _NS
_B_PER_W = _NTOT // _NW
_CH = 32
_NCHUNK = _B_PER_W // _CH

_mesh = plsc.VectorSubcoreMesh(core_axis_name="c", subcore_axis_name="s")


@functools.partial(
    pl.kernel,
    mesh=_mesh,
    out_type=jax.ShapeDtypeStruct((_NTOT, _EMBED), jnp.float32),
    scratch_types=[
        pltpu.VMEM((_B_PER_W,), jnp.int32),
        pltpu.VMEM((2, _CH, _EMBED), jnp.float32),
        pltpu.SemaphoreType.DMA((2,)),
        pltpu.SemaphoreType.DMA((2,)),
    ],
)
def _sc_gather(pe_hbm, idx_hbm, out_hbm, idx_v, rows_v, gsem, ssem):
    wid = lax.axis_index("s") * _NC + lax.axis_index("c")
    base = wid * _B_PER_W
    pltpu.sync_copy(idx_hbm.at[pl.ds(base, _B_PER_W)], idx_v)

    def gather_start(c, b):
        pltpu.make_async_copy(
            pe_hbm.at[idx_v.at[pl.ds(c * _CH, _CH)]], rows_v.at[b], gsem.at[b]
        ).start()

    def gather_wait(b):
        pltpu.make_async_copy(
            pe_hbm.at[pl.ds(0, _CH)], rows_v.at[b], gsem.at[b]
        ).wait()

    def store_start(c, b):
        pltpu.make_async_copy(
            rows_v.at[b], out_hbm.at[pl.ds(base + c * _CH, _CH)], ssem.at[b]
        ).start()

    def store_wait(b):
        pltpu.make_async_copy(
            rows_v.at[b], out_hbm.at[pl.ds(base, _CH)], ssem.at[b]
        ).wait()

    gather_start(0, 0)
    gather_start(1, 1)
    gather_wait(0)
    store_start(0, 0)

    def body(j, _):
        b = j & 1
        nb = 1 - b
        store_wait(nb)
        gather_start(j + 1, nb)
        gather_wait(b)
        store_start(j, b)
        return 0

    lax.fori_loop(1, _NCHUNK - 1, body, 0)

    bl = (_NCHUNK - 1) & 1
    gather_wait(bl)
    store_start(_NCHUNK - 1, bl)
    store_wait(0)
    store_wait(1)


def kernel(positions, pe):
    idx = positions.reshape(-1).astype(jnp.int32)
    out = _sc_gather(pe, idx)
    return out.reshape(_BATCH, _SEQ, _EMBED)
